# baseline (speedup 1.0000x reference)
import functools

import jax
import jax.numpy as jnp
import numpy as np
from jax import lax
from jax.experimental import pallas as pl
from jax.experimental.pallas import tpu as pltpu
from jax.experimental.pallas import tpu_sc as plsc

_ALPHA = 0.3
_BETA = 0.3
_B = 256

_PERM = np.asarray(
    jax.random.permutation(jax.random.fold_in(jax.random.key(42), 1), _B)
).astype(np.int64)

_NC = 2
_NW = 32
_RPW = _B // _NW
_NCH = 56
_CHW = 150528 // _NCH

_row = (np.arange(_NW)[:, None] * _RPW + np.arange(_RPW)[None, :])
_IDXD = (_row[:, None, :] * _NCH + np.arange(_NCH)[None, :, None]).astype(np.int32)
_IDXP = (_PERM[_row][:, None, :] * _NCH + np.arange(_NCH)[None, :, None]).astype(np.int32)


@functools.partial(
    pl.kernel,
    out_type=jax.ShapeDtypeStruct((_B * _NCH, _CHW), jnp.float32),
    mesh=plsc.VectorSubcoreMesh(core_axis_name="c", subcore_axis_name="s"),
    scratch_types=[
        pltpu.VMEM((_NCH, _RPW), jnp.int32),
        pltpu.VMEM((_NCH, _RPW), jnp.int32),
        pltpu.VMEM((16,), jnp.float32),
        pltpu.VMEM((_RPW, _CHW), jnp.float32),
        pltpu.VMEM((_RPW, _CHW), jnp.float32),
        pltpu.VMEM((_RPW, _CHW), jnp.float32),
        pltpu.SemaphoreType.DMA,
        pltpu.SemaphoreType.DMA,
        pltpu.SemaphoreType.DMA,
    ],
)
def _sc_mix(xf_hbm, idxd_hbm, idxp_hbm, lamb_hbm, out_hbm,
            idxd_v, idxp_v, lamb_v, dbuf, pbuf, obuf, semd, semp, semo):
    wid = lax.axis_index("s") * _NC + lax.axis_index("c")
    pltpu.sync_copy(idxd_hbm.at[wid], idxd_v)
    pltpu.sync_copy(idxp_hbm.at[wid], idxp_v)
    pltpu.sync_copy(lamb_hbm, lamb_v)
    lam = lamb_v[...]
    om = 1.0 - lam

    def chunk(c, carry):
        cd = pltpu.async_copy(xf_hbm.at[idxd_v.at[c]], dbuf, semd)
        cp = pltpu.async_copy(xf_hbm.at[idxp_v.at[c]], pbuf, semp)
        cd.wait()
        cp.wait()

        def vloop(v, carry2):
            off = v * 16
            for r in range(_RPW):
                obuf[r, pl.ds(off, 16)] = (
                    lam * dbuf[r, pl.ds(off, 16)]
                    + om * pbuf[r, pl.ds(off, 16)]
                )
            return carry2

        lax.fori_loop(0, _CHW // 16, vloop, 0)
        pltpu.async_copy(obuf, out_hbm.at[idxd_v.at[c]], semo).wait()
        return carry

    lax.fori_loop(0, _NCH, chunk, 0)


def _y_body(idx_ref, lamb_ref, yd_ref, yp_ref, oy_ref):
    lam = lamb_ref[0]
    oy_ref[...] = lam * yd_ref[...] + (1.0 - lam) * yp_ref[...]


def kernel(x, y):
    kl = jax.random.fold_in(jax.random.key(42), 0)
    lamb = jax.random.beta(kl, _ALPHA, _BETA, dtype=jnp.float32)

    B = x.shape[0]
    xf = x.reshape(B * _NCH, _CHW)
    L = y.shape[1]
    yf = y.reshape(B, 1, L)

    mixed_xf = _sc_mix(xf, jnp.asarray(_IDXD), jnp.asarray(_IDXP),
                       jnp.full((16,), lamb, jnp.float32))

    perm_j = jnp.asarray(_PERM.astype(np.int32))
    grid_spec = pltpu.PrefetchScalarGridSpec(
        num_scalar_prefetch=2,
        grid=(B,),
        in_specs=[
            pl.BlockSpec((1, 1, L), lambda i, idx, lam: (i, 0, 0)),
            pl.BlockSpec((1, 1, L), lambda i, idx, lam: (idx[i], 0, 0)),
        ],
        out_specs=pl.BlockSpec((1, 1, L), lambda i, idx, lam: (i, 0, 0)),
    )
    mixed_y = pl.pallas_call(
        _y_body,
        grid_spec=grid_spec,
        out_shape=jax.ShapeDtypeStruct((B, 1, L), jnp.float32),
    )(perm_j, lamb.reshape(1), yf, yf)

    return (mixed_xf.reshape(x.shape), mixed_y.reshape(B, L))


# speedup vs baseline: 1.2161x; 1.2161x over previous
import functools

import jax
import jax.numpy as jnp
import numpy as np
from jax import lax
from jax.experimental import pallas as pl
from jax.experimental.pallas import tpu as pltpu
from jax.experimental.pallas import tpu_sc as plsc

_ALPHA = 0.3
_BETA = 0.3
_B = 256

with jax.default_device(jax.local_devices(backend="cpu")[0]):
    _PERM = np.asarray(
        jax.random.permutation(jax.random.fold_in(jax.random.key(42), 1), _B)
    ).astype(np.int64)

_NC = 2
_NW = 32
_RPW = _B // _NW
_NCH = 84
_CHW = 150528 // _NCH

_row = (np.arange(_NW)[:, None] * _RPW + np.arange(_RPW)[None, :])
_IDXD = (_row[:, None, :] * _NCH + np.arange(_NCH)[None, :, None]).astype(np.int32)
_IDXP = (_PERM[_row][:, None, :] * _NCH + np.arange(_NCH)[None, :, None]).astype(np.int32)


@functools.partial(
    pl.kernel,
    out_type=jax.ShapeDtypeStruct((_B * _NCH, _CHW), jnp.float32),
    mesh=plsc.VectorSubcoreMesh(core_axis_name="c", subcore_axis_name="s"),
    scratch_types=[
        pltpu.VMEM((_NCH, _RPW), jnp.int32),
        pltpu.VMEM((_NCH, _RPW), jnp.int32),
        pltpu.VMEM((16,), jnp.float32),
        pltpu.VMEM((2, _RPW, _CHW), jnp.float32),
        pltpu.VMEM((2, _RPW, _CHW), jnp.float32),
        pltpu.VMEM((2, _RPW, _CHW), jnp.float32),
        pltpu.SemaphoreType.DMA,
        pltpu.SemaphoreType.DMA,
        pltpu.SemaphoreType.DMA,
    ],
)
def _sc_mix(xf_hbm, idxd_hbm, idxp_hbm, lamb_hbm, out_hbm,
            idxd_v, idxp_v, lamb_v, dbuf, pbuf, obuf, semd, semp, semo):
    wid = lax.axis_index("s") * _NC + lax.axis_index("c")
    pltpu.sync_copy(idxd_hbm.at[wid], idxd_v)
    pltpu.sync_copy(idxp_hbm.at[wid], idxp_v)
    pltpu.sync_copy(lamb_hbm, lamb_v)
    lam = lamb_v[...]
    om = 1.0 - lam

    def start_gather(c, b):
        pltpu.async_copy(xf_hbm.at[idxd_v.at[c]], dbuf.at[b], semd)
        pltpu.async_copy(xf_hbm.at[idxp_v.at[c]], pbuf.at[b], semp)

    def wait_gather(b):
        pltpu.make_async_copy(xf_hbm.at[idxd_v.at[0]], dbuf.at[b], semd).wait()
        pltpu.make_async_copy(xf_hbm.at[idxp_v.at[0]], pbuf.at[b], semp).wait()

    def wait_scatter(b):
        pltpu.make_async_copy(obuf.at[b], out_hbm.at[idxd_v.at[0]], semo).wait()

    def compute(b):
        def vloop(v, carry2):
            off = v * 16
            for r in range(_RPW):
                obuf[b, r, pl.ds(off, 16)] = (
                    lam * dbuf[b, r, pl.ds(off, 16)]
                    + om * pbuf[b, r, pl.ds(off, 16)]
                )
            return carry2

        lax.fori_loop(0, _CHW // 16, vloop, 0)

    def start_scatter(c, b):
        pltpu.async_copy(obuf.at[b], out_hbm.at[idxd_v.at[c]], semo)

    start_gather(0, 0)
    start_gather(1, 1)
    for b in range(2):
        wait_gather(b)
        compute(b)
        start_scatter(b, b)
        start_gather(b + 2, b)

    def outer(kk, carry):
        for b in range(2):
            k = 2 * kk + b
            wait_gather(b)
            wait_scatter(b)
            compute(b)
            start_scatter(k, b)

            @pl.when(k + 2 < _NCH)
            def _():
                start_gather(k + 2, b)
        return carry

    lax.fori_loop(1, _NCH // 2, outer, 0)
    wait_scatter(0)
    wait_scatter(1)


def _y_body(idx_ref, lamb_ref, yd_ref, yp_ref, oy_ref):
    lam = lamb_ref[0]
    oy_ref[...] = lam * yd_ref[...] + (1.0 - lam) * yp_ref[...]


def kernel(x, y):
    kl = jax.random.fold_in(jax.random.key(42), 0)
    lamb = jax.random.beta(kl, _ALPHA, _BETA, dtype=jnp.float32)

    B = x.shape[0]
    xf = x.reshape(B * _NCH, _CHW)
    L = y.shape[1]
    yf = y.reshape(B, 1, L)

    mixed_xf = _sc_mix(xf, jnp.asarray(_IDXD), jnp.asarray(_IDXP),
                       jnp.full((16,), lamb, jnp.float32))

    perm_j = jnp.asarray(_PERM.astype(np.int32))
    grid_spec = pltpu.PrefetchScalarGridSpec(
        num_scalar_prefetch=2,
        grid=(B,),
        in_specs=[
            pl.BlockSpec((1, 1, L), lambda i, idx, lam: (i, 0, 0)),
            pl.BlockSpec((1, 1, L), lambda i, idx, lam: (idx[i], 0, 0)),
        ],
        out_specs=pl.BlockSpec((1, 1, L), lambda i, idx, lam: (i, 0, 0)),
    )
    mixed_y = pl.pallas_call(
        _y_body,
        grid_spec=grid_spec,
        out_shape=jax.ShapeDtypeStruct((B, 1, L), jnp.float32),
    )(perm_j, lamb.reshape(1), yf, yf)

    return (mixed_xf.reshape(x.shape), mixed_y.reshape(B, L))


# speedup vs baseline: 1.2323x; 1.0133x over previous
import functools

import jax
import jax.numpy as jnp
import numpy as np
from jax import lax
from jax.experimental import pallas as pl
from jax.experimental.pallas import tpu as pltpu
from jax.experimental.pallas import tpu_sc as plsc

_ALPHA = 0.3
_BETA = 0.3
_B = 256

with jax.default_device(jax.local_devices(backend="cpu")[0]):
    _PERM = np.asarray(
        jax.random.permutation(jax.random.fold_in(jax.random.key(42), 1), _B)
    ).astype(np.int64)

_NC = 2
_NW = 32
_RPW = _B // _NW
_NCH = 84
_CHW = 150528 // _NCH

_row = (np.arange(_NW)[:, None] * _RPW + np.arange(_RPW)[None, :])
_IDXD = (_row[:, None, :] * _NCH + np.arange(_NCH)[None, :, None]).astype(np.int32)
_IDXP = (_PERM[_row][:, None, :] * _NCH + np.arange(_NCH)[None, :, None]).astype(np.int32)


@functools.partial(
    pl.kernel,
    out_type=jax.ShapeDtypeStruct((_B * _NCH, _CHW), jnp.float32),
    mesh=plsc.VectorSubcoreMesh(core_axis_name="c", subcore_axis_name="s"),
    compiler_params=pltpu.CompilerParams(use_tc_tiling_on_sc=False),
    scratch_types=[
        pltpu.VMEM((_NCH, _RPW), jnp.int32),
        pltpu.VMEM((_NCH, _RPW), jnp.int32),
        pltpu.VMEM((16,), jnp.float32),
        pltpu.VMEM((2, _RPW, _CHW), jnp.float32),
        pltpu.VMEM((2, _RPW, _CHW), jnp.float32),
        pltpu.VMEM((2, _RPW, _CHW), jnp.float32),
        pltpu.SemaphoreType.DMA,
        pltpu.SemaphoreType.DMA,
        pltpu.SemaphoreType.DMA,
    ],
)
def _sc_mix(xf_hbm, idxd_hbm, idxp_hbm, lamb_hbm, out_hbm,
            idxd_v, idxp_v, lamb_v, dbuf, pbuf, obuf, semd, semp, semo):
    wid = lax.axis_index("s") * _NC + lax.axis_index("c")
    pltpu.sync_copy(idxd_hbm.at[wid], idxd_v)
    pltpu.sync_copy(idxp_hbm.at[wid], idxp_v)
    pltpu.sync_copy(lamb_hbm, lamb_v)
    lam = lamb_v[...]
    om = 1.0 - lam

    def start_gather(c, b):
        pltpu.async_copy(xf_hbm.at[idxd_v.at[c]], dbuf.at[b], semd)
        pltpu.async_copy(xf_hbm.at[idxp_v.at[c]], pbuf.at[b], semp)

    def wait_gather(b):
        pltpu.make_async_copy(xf_hbm.at[idxd_v.at[0]], dbuf.at[b], semd).wait()
        pltpu.make_async_copy(xf_hbm.at[idxp_v.at[0]], pbuf.at[b], semp).wait()

    def wait_scatter(b):
        pltpu.make_async_copy(obuf.at[b], out_hbm.at[idxd_v.at[0]], semo).wait()

    def compute(b):
        def vloop(v, carry2):
            off = v * 16
            for r in range(_RPW):
                obuf[b, r, pl.ds(off, 16)] = (
                    lam * dbuf[b, r, pl.ds(off, 16)]
                    + om * pbuf[b, r, pl.ds(off, 16)]
                )
            return carry2

        lax.fori_loop(0, _CHW // 16, vloop, 0)

    def start_scatter(c, b):
        pltpu.async_copy(obuf.at[b], out_hbm.at[idxd_v.at[c]], semo)

    start_gather(0, 0)
    start_gather(1, 1)
    for b in range(2):
        wait_gather(b)
        compute(b)
        start_scatter(b, b)
        start_gather(b + 2, b)

    def outer(kk, carry):
        for b in range(2):
            k = 2 * kk + b
            wait_gather(b)
            wait_scatter(b)
            compute(b)
            start_scatter(k, b)

            @pl.when(k + 2 < _NCH)
            def _():
                start_gather(k + 2, b)
        return carry

    lax.fori_loop(1, _NCH // 2, outer, 0)
    wait_scatter(0)
    wait_scatter(1)


def _y_body(idx_ref, lamb_ref, yd_ref, yp_ref, oy_ref):
    lam = lamb_ref[0]
    oy_ref[...] = lam * yd_ref[...] + (1.0 - lam) * yp_ref[...]


def kernel(x, y):
    kl = jax.random.fold_in(jax.random.key(42), 0)
    lamb = jax.random.beta(kl, _ALPHA, _BETA, dtype=jnp.float32)

    B = x.shape[0]
    xf = x.reshape(B * _NCH, _CHW)
    L = y.shape[1]
    yf = y.reshape(B, 1, L)

    mixed_xf = _sc_mix(xf, jnp.asarray(_IDXD), jnp.asarray(_IDXP),
                       jnp.full((16,), lamb, jnp.float32))

    perm_j = jnp.asarray(_PERM.astype(np.int32))
    grid_spec = pltpu.PrefetchScalarGridSpec(
        num_scalar_prefetch=2,
        grid=(B,),
        in_specs=[
            pl.BlockSpec((1, 1, L), lambda i, idx, lam: (i, 0, 0)),
            pl.BlockSpec((1, 1, L), lambda i, idx, lam: (idx[i], 0, 0)),
        ],
        out_specs=pl.BlockSpec((1, 1, L), lambda i, idx, lam: (i, 0, 0)),
    )
    mixed_y = pl.pallas_call(
        _y_body,
        grid_spec=grid_spec,
        out_shape=jax.ShapeDtypeStruct((B, 1, L), jnp.float32),
    )(perm_j, lamb.reshape(1), yf, yf)

    return (mixed_xf.reshape(x.shape), mixed_y.reshape(B, L))


# speedup vs baseline: 1.7330x; 1.4063x over previous
import jax
import jax.numpy as jnp
import numpy as np
from jax.experimental import pallas as pl
from jax.experimental.pallas import tpu as pltpu

_ALPHA = 0.3
_BETA = 0.3
_B = 256

with jax.default_device(jax.local_devices(backend="cpu")[0]):
    _PERM = np.asarray(
        jax.random.permutation(jax.random.fold_in(jax.random.key(42), 1), _B)
    ).astype(np.int64)

_visited = [False] * _B
_order, _nxt = [], []
for _s in range(_B):
    if not _visited[_s]:
        _c = _s
        while not _visited[_c]:
            _visited[_c] = True
            _order.append(_c)
            _nxt.append(int(_PERM[_c]))
            _c = int(_PERM[_c])

_ORDER = np.asarray(_order, dtype=np.int32)
_NXT = np.asarray(_nxt, dtype=np.int32)
_E_IDX = np.where(np.arange(_B) % 2 == 0, _ORDER, _NXT).astype(np.int32)
_O_IDX = np.where(np.arange(_B) % 2 == 0, _NXT, _ORDER).astype(np.int32)


def _mix_body(e_idx, o_idx, ord_idx, nxt_idx, lamb_ref,
              xe_ref, xo_ref, yd_ref, yp_ref, ox_ref, oy_ref):
    lam = lamb_ref[0]
    k = pl.program_id(0)

    @pl.when(k % 2 == 0)
    def _():
        ox_ref[...] = lam * xe_ref[...] + (1.0 - lam) * xo_ref[...]

    @pl.when(k % 2 == 1)
    def _():
        ox_ref[...] = lam * xo_ref[...] + (1.0 - lam) * xe_ref[...]

    oy_ref[...] = lam * yd_ref[...] + (1.0 - lam) * yp_ref[...]


def kernel(x, y):
    kl = jax.random.fold_in(jax.random.key(42), 0)
    lamb = jax.random.beta(kl, _ALPHA, _BETA, dtype=jnp.float32)
    B, C, H, W = x.shape
    L = y.shape[1]
    yf = y.reshape(B, 1, L)

    grid_spec = pltpu.PrefetchScalarGridSpec(
        num_scalar_prefetch=5,
        grid=(B,),
        in_specs=[
            pl.BlockSpec((1, C, H, W), lambda k, e, o, od, nx, lam: (e[k], 0, 0, 0)),
            pl.BlockSpec((1, C, H, W), lambda k, e, o, od, nx, lam: (o[k], 0, 0, 0)),
            pl.BlockSpec((1, 1, L), lambda k, e, o, od, nx, lam: (od[k], 0, 0)),
            pl.BlockSpec((1, 1, L), lambda k, e, o, od, nx, lam: (nx[k], 0, 0)),
        ],
        out_specs=[
            pl.BlockSpec((1, C, H, W), lambda k, e, o, od, nx, lam: (od[k], 0, 0, 0)),
            pl.BlockSpec((1, 1, L), lambda k, e, o, od, nx, lam: (od[k], 0, 0)),
        ],
    )

    mixed_x, mixed_y = pl.pallas_call(
        _mix_body,
        grid_spec=grid_spec,
        out_shape=[
            jax.ShapeDtypeStruct((B, C, H, W), jnp.float32),
            jax.ShapeDtypeStruct((B, 1, L), jnp.float32),
        ],
    )(_E_IDX, _O_IDX, _ORDER, _NXT, lamb.reshape(1), x, x, yf, yf)

    return (mixed_x, mixed_y.reshape(B, L))


# speedup vs baseline: 2.3668x; 1.3657x over previous
import functools

import jax
import jax.numpy as jnp
import numpy as np
from jax import lax
from jax.experimental import pallas as pl
from jax.experimental.pallas import tpu as pltpu
from jax.experimental.pallas import tpu_sc as plsc

_ALPHA = 0.3
_BETA = 0.3
_B = 256

with jax.default_device(jax.local_devices(backend="cpu")[0]):
    _PERM = np.asarray(
        jax.random.permutation(jax.random.fold_in(jax.random.key(42), 1), _B)
    ).astype(np.int64)

_PMAT = np.zeros((_B, _B), dtype=np.float32)
_PMAT[np.arange(_B), _PERM] = 1.0

_NC = 2
_NW = 32
_Q = 3072
_QPW = _Q // _NW
_SL = 56
_LN = 224

_qb = np.arange(_Q) // 12
_qr = np.arange(_Q) % 12
_PQ = (_PERM[_qb] * 12 + _qr).astype(np.int32)
_IDXP = np.repeat(_PQ.reshape(_NW, _QPW, 1), 16, axis=2)


@functools.partial(
    pl.kernel,
    out_type=jax.ShapeDtypeStruct((_Q, _SL, _LN), jnp.float32),
    mesh=plsc.VectorSubcoreMesh(core_axis_name="c", subcore_axis_name="s"),
    scratch_types=[
        pltpu.VMEM((_QPW, 16), jnp.int32),
        pltpu.VMEM((16,), jnp.float32),
        pltpu.VMEM((2, _SL, _LN), jnp.float32),
        pltpu.VMEM((2, _SL, _LN), jnp.float32),
        pltpu.VMEM((2, _SL, _LN), jnp.float32),
        pltpu.SemaphoreType.DMA,
        pltpu.SemaphoreType.DMA,
        pltpu.SemaphoreType.DMA,
    ],
)
def _sc_mix(xq_hbm, idxp_hbm, lamb_hbm, out_hbm,
            idxp_v, lamb_v, dbuf, pbuf, obuf, semd, semp, semo):
    wid = lax.axis_index("s") * _NC + lax.axis_index("c")
    base = wid * _QPW
    pltpu.sync_copy(idxp_hbm.at[wid], idxp_v)
    pltpu.sync_copy(lamb_hbm, lamb_v)
    lam = lamb_v[...]
    om = 1.0 - lam

    def perm_row(j):
        return idxp_v[j, :][0]

    def start_gather(j, b):
        pltpu.async_copy(xq_hbm.at[base + j], dbuf.at[b], semd)
        pltpu.async_copy(xq_hbm.at[perm_row(j)], pbuf.at[b], semp)

    def wait_gather(b):
        pltpu.make_async_copy(xq_hbm.at[0], dbuf.at[b], semd).wait()
        pltpu.make_async_copy(xq_hbm.at[0], pbuf.at[b], semp).wait()

    def wait_scatter(b):
        pltpu.make_async_copy(obuf.at[b], out_hbm.at[0], semo).wait()

    def compute(b):
        def vloop(s, carry):
            for v in range(_LN // 16):
                off = v * 16
                obuf[b, s, pl.ds(off, 16)] = (
                    lam * dbuf[b, s, pl.ds(off, 16)]
                    + om * pbuf[b, s, pl.ds(off, 16)]
                )
            return carry

        lax.fori_loop(0, _SL, vloop, 0)

    def start_scatter(j, b):
        pltpu.async_copy(obuf.at[b], out_hbm.at[base + j], semo)

    start_gather(0, 0)
    start_gather(1, 1)
    for b in range(2):
        wait_gather(b)
        compute(b)
        start_scatter(b, b)
        start_gather(b + 2, b)

    def outer(kk, carry):
        for b in range(2):
            j = 2 * kk + b
            wait_gather(b)
            wait_scatter(b)
            compute(b)
            start_scatter(j, b)

            @pl.when(j + 2 < _QPW)
            def _():
                start_gather(j + 2, b)
        return carry

    lax.fori_loop(1, _QPW // 2, outer, 0)
    wait_scatter(0)
    wait_scatter(1)


def _y_body(m_ref, y_ref, oy_ref):
    oy_ref[...] = jnp.dot(m_ref[...], y_ref[...],
                          preferred_element_type=jnp.float32)


def kernel(x, y):
    kl = jax.random.fold_in(jax.random.key(42), 0)
    lamb = jax.random.beta(kl, _ALPHA, _BETA, dtype=jnp.float32)

    B, C, H, W = x.shape
    xq = x.reshape(_Q, _SL, _LN)
    L = y.shape[1]

    mixed_xq = _sc_mix(xq, jnp.asarray(_IDXP),
                       jnp.full((16,), lamb, jnp.float32))

    mmat = lamb * jnp.eye(B, dtype=jnp.float32) \
        + (1.0 - lamb) * jnp.asarray(_PMAT)
    mixed_y = pl.pallas_call(
        _y_body,
        out_shape=jax.ShapeDtypeStruct((B, L), jnp.float32),
    )(mmat, y)

    return (mixed_xq.reshape(B, C, H, W), mixed_y)
